# baseline (speedup 1.0000x reference)
import jax
import jax.numpy as jnp
from jax import lax
from jax.experimental import pallas as pl
from jax.experimental.pallas import tpu as pltpu
from jax.experimental.pallas import tpu_sc as plsc

NC, NS, L = 2, 16, 16
NW = NC * NS
K = 48
F32 = jnp.float32


def _ln(x, g, b, eps=1e-5):
    mu = jnp.mean(x, axis=-1, keepdims=True)
    var = jnp.mean((x - mu) ** 2, axis=-1, keepdims=True)
    return (x - mu) * lax.rsqrt(var + eps) * g + b


def _acc_rows(N):
    degbase = -(-(N + 1) // (NS * 8)) * (NS * 8)
    degrows = -(-N // 128)
    np2 = -(-(degbase + degrows) // (NS * 8)) * (NS * 8)
    return degbase, degrows, np2


def _make_edge_scatter(N, D, CPW, k):
    _, _, Np2 = _acc_rows(N)
    RPT = Np2 // NS

    mesh = plsc.VectorSubcoreMesh(core_axis_name="c", subcore_axis_name="s")
    scratch = [
        pltpu.VMEM((k,), jnp.int32),
        pltpu.VMEM((k,), jnp.int32),
        pltpu.VMEM((k,), jnp.int32),
        pltpu.VMEM((k, D), F32),
        pltpu.VMEM((k, D), F32),
        pltpu.VMEM_SHARED((Np2, D), F32),
        pltpu.SemaphoreType.DMA,
    ]

    def body(src_hbm, dst_hbm, et_hbm, xtab, rtab, zpre, out_pre,
             src_i, dst_i, et_i, xrows, rrows, pre_sh, sem):
        c = lax.axis_index("c")
        s = lax.axis_index("s")
        base = (c * NS + s) * CPW
        rows = pl.ds(s * RPT, RPT)

        pltpu.sync_copy(zpre, pre_sh.at[rows])
        plsc.subcore_barrier()

        def chunk(j, carry):
            sl_e = pl.ds((base + j) * k, k)
            pltpu.sync_copy(src_hbm.at[sl_e], src_i)
            pltpu.sync_copy(dst_hbm.at[sl_e], dst_i)
            pltpu.sync_copy(et_hbm.at[sl_e], et_i)
            gx = pltpu.async_copy(xtab.at[src_i], xrows, sem)
            gr = pltpu.async_copy(rtab.at[et_i], rrows, sem)
            gx.wait()
            gr.wait()

            def edge(i, carry2):
                for q in range(D // L):
                    sl = pl.ds(q * L, L)
                    xrows[i, sl] = xrows[i, sl] * rrows[i, sl]
                return carry2

            lax.fori_loop(0, k, edge, 0)
            pltpu.sync_copy(xrows, pre_sh.at[dst_i], add=True)
            return carry

        lax.fori_loop(0, CPW, chunk, 0)
        plsc.subcore_barrier()
        pltpu.sync_copy(pre_sh.at[rows], out_pre.at[c].at[rows])

    return pl.kernel(body, out_type=jax.ShapeDtypeStruct((NC, Np2, D), F32),
                     mesh=mesh, scratch_types=tuple(scratch))


def _time_body(tin, Wt, bt, g, b, out):
    y = jnp.dot(tin[...], Wt[...], preferred_element_type=F32) + bt[...]
    out[...] = _ln(y, g[...], b[...])


def _rel_body(r, Wr1, gr1, br1, Wr2, gr2, br2, r1_out, r2_out):
    r1 = jnp.tanh(_ln(jnp.dot(r[...], Wr1[...], preferred_element_type=F32),
                      gr1[...], br1[...]))
    r1_out[...] = r1
    r2_out[...] = jnp.tanh(_ln(jnp.dot(r1, Wr2[...], preferred_element_type=F32),
                               gr2[...], br2[...]))


def _layer_body(pre0, pre1, d0, d1, x, t, W, Wl, b, g, be, out):
    pre = pre0[...] + pre1[...]
    deg = d0[...] + d1[...]
    agg = jnp.dot(pre, W[...], preferred_element_type=F32)
    agg = agg / jnp.maximum(deg, 1.0)
    y = agg + jnp.dot(x[...], Wl[...], preferred_element_type=F32)
    y = y + b[...] + t[...]
    out[...] = jnp.tanh(_ln(y, g[...], be[...]))


def _row_spec(bt, d):
    return pl.BlockSpec((bt, d), lambda i: (i, 0))


def _full_spec(shape):
    return pl.BlockSpec(shape, lambda i: tuple(0 for _ in shape))


def _pad_edges(srcv, dstv, etv, N, k):
    E2 = srcv.shape[0]
    cpw = -(-E2 // (NW * k))
    ep = NW * cpw * k
    return (jnp.pad(srcv, (0, ep - E2)),
            jnp.pad(dstv, (0, ep - E2), constant_values=N),
            jnp.pad(etv, (0, ep - E2)),
            cpw)


def kernel(edge_index, edge_type, time_emds, ent_emds, rel_emds, Wt, bt, g_t,
           b_tl, W1, Wl1, Wr1, b1, ge1, be1, gr1, br1, W2, Wl2, Wr2, b2, ge2,
           be2, gr2, br2):
    N, D = ent_emds.shape
    E = edge_type.shape[0]
    R = rel_emds.shape[0]
    DEGBASE, DEGROWS, Np2 = _acc_rows(N)

    src = edge_index[0]
    dst = edge_index[1]

    syn_src = N + jnp.bitwise_and(dst, 127)
    syn_dst = DEGBASE + jnp.right_shift(dst, 7)
    syn_et = jnp.full((E,), R, jnp.int32)
    src1, dst1, et1, CPW1 = _pad_edges(
        jnp.concatenate([src, syn_src]), jnp.concatenate([dst, syn_dst]),
        jnp.concatenate([edge_type, syn_et]), N, K)
    src2, dst2, et2, CPW2 = _pad_edges(src, dst, edge_type, N, K)

    xtab1 = jnp.concatenate([ent_emds, jnp.eye(128, dtype=F32)])
    rtab1 = jnp.concatenate([rel_emds, jnp.ones((1, D), F32)])
    zpre = jnp.zeros((Np2 // NS, D), F32)

    BT = 2000
    grid = N // BT
    t = pl.pallas_call(
        _time_body,
        grid=(grid,),
        in_specs=[_row_spec(BT, D), _full_spec((D, D)), _full_spec((1, D)),
                  _full_spec((1, D)), _full_spec((1, D))],
        out_specs=_row_spec(BT, D),
        out_shape=jax.ShapeDtypeStruct((N, D), F32),
    )(time_emds, Wt, bt.reshape(1, D), g_t.reshape(1, D), b_tl.reshape(1, D))

    r1, r2 = pl.pallas_call(
        _rel_body,
        in_specs=[pl.BlockSpec((R, D), lambda: (0, 0)),
                  pl.BlockSpec((D, D), lambda: (0, 0))] +
                 [pl.BlockSpec((1, D), lambda: (0, 0))] * 2 +
                 [pl.BlockSpec((D, D), lambda: (0, 0))] +
                 [pl.BlockSpec((1, D), lambda: (0, 0))] * 2,
        out_specs=[pl.BlockSpec((R, D), lambda: (0, 0))] * 2,
        out_shape=[jax.ShapeDtypeStruct((R, D), F32)] * 2,
    )(rel_emds, Wr1, gr1.reshape(1, D), br1.reshape(1, D),
      Wr2, gr2.reshape(1, D), br2.reshape(1, D))

    layer_call = pl.pallas_call(
        _layer_body,
        grid=(grid,),
        in_specs=[_row_spec(BT, D), _row_spec(BT, D),
                  _row_spec(BT, 1), _row_spec(BT, 1),
                  _row_spec(BT, D), _row_spec(BT, D),
                  _full_spec((D, D)), _full_spec((D, D)),
                  _full_spec((1, D)), _full_spec((1, D)), _full_spec((1, D))],
        out_specs=_row_spec(BT, D),
        out_shape=jax.ShapeDtypeStruct((N, D), F32),
    )

    scat1 = _make_edge_scatter(N, D, CPW1, K)
    acc1 = scat1(src1, dst1, et1, xtab1, rtab1, zpre)
    d0 = acc1[0, DEGBASE:DEGBASE + DEGROWS].reshape(DEGROWS * 128, 1)[:N]
    d1 = acc1[1, DEGBASE:DEGBASE + DEGROWS].reshape(DEGROWS * 128, 1)[:N]
    x1 = layer_call(acc1[0], acc1[1], d0, d1, ent_emds, t, W1, Wl1,
                    b1.reshape(1, D), ge1.reshape(1, D), be1.reshape(1, D))

    scat2 = _make_edge_scatter(N, D, CPW2, K)
    acc2 = scat2(src2, dst2, et2, x1, r1, zpre)
    x2 = layer_call(acc2[0], acc2[1], d0, d1, x1, t, W2, Wl2,
                    b2.reshape(1, D), ge2.reshape(1, D), be2.reshape(1, D))

    return (x2, r2)


# speedup vs baseline: 1.0224x; 1.0224x over previous
import jax
import jax.numpy as jnp
from jax import lax
from jax.experimental import pallas as pl
from jax.experimental.pallas import tpu as pltpu
from jax.experimental.pallas import tpu_sc as plsc

NC, NS, L = 2, 16, 16
NW = NC * NS
K = 48
F32 = jnp.float32


def _ln(x, g, b, eps=1e-5):
    mu = jnp.mean(x, axis=-1, keepdims=True)
    var = jnp.mean((x - mu) ** 2, axis=-1, keepdims=True)
    return (x - mu) * lax.rsqrt(var + eps) * g + b


def _acc_rows(N):
    degbase = -(-(N + 1) // (NS * 8)) * (NS * 8)
    degrows = -(-N // 128)
    np2 = -(-(degbase + degrows) // (NS * 8)) * (NS * 8)
    return degbase, degrows, np2


def _make_edge_scatter(N, D, CPW, k):
    _, _, Np2 = _acc_rows(N)
    RPT = Np2 // NS
    assert CPW % 6 == 0

    mesh = plsc.VectorSubcoreMesh(core_axis_name="c", subcore_axis_name="s")
    scratch = (
        [pltpu.VMEM((3, k), jnp.int32)] * 3 +
        [pltpu.VMEM((k, D), F32)] * 2 +
        [pltpu.VMEM((k, D), F32)] * 2 +
        [pltpu.VMEM_SHARED((Np2, D), F32)] +
        [pltpu.SemaphoreType.DMA] * 3 +
        [pltpu.SemaphoreType.DMA] * 2
    )

    def body(eidx, xtab, rtab, zpre, out_pre,
             idx0, idx1, idx2, xr0, xr1, rr0, rr1, pre_sh,
             si0, si1, si2, sg0, sg1):
        idx = (idx0, idx1, idx2)
        xr = (xr0, xr1)
        rr = (rr0, rr1)
        si = (si0, si1, si2)
        sg = (sg0, sg1)
        c = lax.axis_index("c")
        s = lax.axis_index("s")
        base = (c * NS + s) * CPW
        rows = pl.ds(s * RPT, RPT)

        pltpu.sync_copy(zpre, pre_sh.at[rows])

        def issue_idx(j, t):
            pltpu.async_copy(eidx.at[base + j], idx[t], si[t])

        def wait_idx(t):
            pltpu.make_async_copy(eidx.at[base], idx[t], si[t]).wait()

        def issue_gather(t, b):
            pltpu.async_copy(xtab.at[idx[t].at[0]], xr[b], sg[b])
            pltpu.async_copy(rtab.at[idx[t].at[2]], rr[b], sg[b])

        def wait_gather(t, b):
            pltpu.make_async_copy(xtab.at[idx[t].at[0]], xr[b], sg[b]).wait()
            pltpu.make_async_copy(rtab.at[idx[t].at[2]], rr[b], sg[b]).wait()

        issue_idx(0, 0)
        issue_idx(1, 1)
        wait_idx(0)
        issue_gather(0, 0)
        plsc.subcore_barrier()

        def superstep(jj, carry):
            for u in range(6):
                j = jj * 6 + u
                b = u % 2
                t = u % 3
                tn = (u + 1) % 3
                t2 = (u + 2) % 3

                @pl.when(j + 1 < CPW)
                def _():
                    wait_idx(tn)
                    issue_gather(tn, 1 - b)

                wait_gather(t, b)

                @pl.when(j + 2 < CPW)
                def _():
                    issue_idx(j + 2, t2)

                def edge(i, carry2):
                    for q in range(D // L):
                        sl = pl.ds(q * L, L)
                        xr[b][i, sl] = xr[b][i, sl] * rr[b][i, sl]
                    return carry2

                lax.fori_loop(0, k, edge, 0)
                pltpu.sync_copy(xr[b], pre_sh.at[idx[t].at[1]], add=True)
            return carry

        lax.fori_loop(0, CPW // 6, superstep, 0)
        plsc.subcore_barrier()
        pltpu.sync_copy(pre_sh.at[rows], out_pre.at[c].at[rows])

    return pl.kernel(body, out_type=jax.ShapeDtypeStruct((NC, Np2, D), F32),
                     mesh=mesh, scratch_types=tuple(scratch))


def _time_body(tin, Wt, bt, g, b, out):
    y = jnp.dot(tin[...], Wt[...], preferred_element_type=F32) + bt[...]
    out[...] = _ln(y, g[...], b[...])


def _rel_body(r, Wr1, gr1, br1, Wr2, gr2, br2, r1_out, r2_out):
    r1 = jnp.tanh(_ln(jnp.dot(r[...], Wr1[...], preferred_element_type=F32),
                      gr1[...], br1[...]))
    r1_out[...] = r1
    r2_out[...] = jnp.tanh(_ln(jnp.dot(r1, Wr2[...], preferred_element_type=F32),
                               gr2[...], br2[...]))


def _layer_body(pre0, pre1, d0, d1, x, t, W, Wl, b, g, be, out):
    pre = pre0[...] + pre1[...]
    deg = d0[...] + d1[...]
    agg = jnp.dot(pre, W[...], preferred_element_type=F32)
    agg = agg / jnp.maximum(deg, 1.0)
    y = agg + jnp.dot(x[...], Wl[...], preferred_element_type=F32)
    y = y + b[...] + t[...]
    out[...] = jnp.tanh(_ln(y, g[...], be[...]))


def _row_spec(bt, d):
    return pl.BlockSpec((bt, d), lambda i: (i, 0))


def _full_spec(shape):
    return pl.BlockSpec(shape, lambda i: tuple(0 for _ in shape))


def _pad_edges(srcv, dstv, etv, N, k):
    E2 = srcv.shape[0]
    cpw = 6 * (-(-E2 // (NW * k * 6)))
    ep = NW * cpw * k
    src = jnp.pad(srcv, (0, ep - E2)).reshape(NW * cpw, 1, k)
    dstp = jnp.pad(dstv, (0, ep - E2), constant_values=N).reshape(NW * cpw, 1, k)
    et = jnp.pad(etv, (0, ep - E2)).reshape(NW * cpw, 1, k)
    return jnp.concatenate([src, dstp, et], axis=1), cpw


def kernel(edge_index, edge_type, time_emds, ent_emds, rel_emds, Wt, bt, g_t,
           b_tl, W1, Wl1, Wr1, b1, ge1, be1, gr1, br1, W2, Wl2, Wr2, b2, ge2,
           be2, gr2, br2):
    N, D = ent_emds.shape
    E = edge_type.shape[0]
    R = rel_emds.shape[0]
    DEGBASE, DEGROWS, Np2 = _acc_rows(N)

    src = edge_index[0]
    dst = edge_index[1]

    syn_src = N + jnp.bitwise_and(dst, 127)
    syn_dst = DEGBASE + jnp.right_shift(dst, 7)
    syn_et = jnp.full((E,), R, jnp.int32)
    eidx1, CPW1 = _pad_edges(
        jnp.concatenate([src, syn_src]), jnp.concatenate([dst, syn_dst]),
        jnp.concatenate([edge_type, syn_et]), N, K)
    eidx2, CPW2 = _pad_edges(src, dst, edge_type, N, K)

    xtab1 = jnp.concatenate([ent_emds, jnp.eye(128, dtype=F32)])
    rtab1 = jnp.concatenate([rel_emds, jnp.ones((1, D), F32)])
    zpre = jnp.zeros((Np2 // NS, D), F32)

    BT = 2000
    grid = N // BT
    t = pl.pallas_call(
        _time_body,
        grid=(grid,),
        in_specs=[_row_spec(BT, D), _full_spec((D, D)), _full_spec((1, D)),
                  _full_spec((1, D)), _full_spec((1, D))],
        out_specs=_row_spec(BT, D),
        out_shape=jax.ShapeDtypeStruct((N, D), F32),
    )(time_emds, Wt, bt.reshape(1, D), g_t.reshape(1, D), b_tl.reshape(1, D))

    r1, r2 = pl.pallas_call(
        _rel_body,
        in_specs=[pl.BlockSpec((R, D), lambda: (0, 0)),
                  pl.BlockSpec((D, D), lambda: (0, 0))] +
                 [pl.BlockSpec((1, D), lambda: (0, 0))] * 2 +
                 [pl.BlockSpec((D, D), lambda: (0, 0))] +
                 [pl.BlockSpec((1, D), lambda: (0, 0))] * 2,
        out_specs=[pl.BlockSpec((R, D), lambda: (0, 0))] * 2,
        out_shape=[jax.ShapeDtypeStruct((R, D), F32)] * 2,
    )(rel_emds, Wr1, gr1.reshape(1, D), br1.reshape(1, D),
      Wr2, gr2.reshape(1, D), br2.reshape(1, D))

    layer_call = pl.pallas_call(
        _layer_body,
        grid=(grid,),
        in_specs=[_row_spec(BT, D), _row_spec(BT, D),
                  _row_spec(BT, 1), _row_spec(BT, 1),
                  _row_spec(BT, D), _row_spec(BT, D),
                  _full_spec((D, D)), _full_spec((D, D)),
                  _full_spec((1, D)), _full_spec((1, D)), _full_spec((1, D))],
        out_specs=_row_spec(BT, D),
        out_shape=jax.ShapeDtypeStruct((N, D), F32),
    )

    scat1 = _make_edge_scatter(N, D, CPW1, K)
    acc1 = scat1(eidx1, xtab1, rtab1, zpre)
    d0 = acc1[0, DEGBASE:DEGBASE + DEGROWS].reshape(DEGROWS * 128, 1)[:N]
    d1 = acc1[1, DEGBASE:DEGBASE + DEGROWS].reshape(DEGROWS * 128, 1)[:N]
    x1 = layer_call(acc1[0], acc1[1], d0, d1, ent_emds, t, W1, Wl1,
                    b1.reshape(1, D), ge1.reshape(1, D), be1.reshape(1, D))

    scat2 = _make_edge_scatter(N, D, CPW2, K)
    acc2 = scat2(eidx2, x1, r1, zpre)
    x2 = layer_call(acc2[0], acc2[1], d0, d1, x1, t, W2, Wl2,
                    b2.reshape(1, D), ge2.reshape(1, D), be2.reshape(1, D))

    return (x2, r2)


# speedup vs baseline: 13.4828x; 13.1875x over previous
import jax
import jax.numpy as jnp
from jax import lax
from jax.experimental import pallas as pl
from jax.experimental.pallas import tpu as pltpu
from jax.experimental.pallas import tpu_sc as plsc

NC, NS, L = 2, 16, 16
NW = NC * NS
K = 48
F32 = jnp.float32


def _ln(x, g, b, eps=1e-5):
    mu = jnp.mean(x, axis=-1, keepdims=True)
    var = jnp.mean((x - mu) ** 2, axis=-1, keepdims=True)
    return (x - mu) * lax.rsqrt(var + eps) * g + b


def _acc_rows(N):
    degbase = -(-(N + 1) // (NS * 8)) * (NS * 8)
    degrows = -(-N // 128)
    np2 = -(-(degbase + degrows) // (NS * 8)) * (NS * 8)
    return degbase, degrows, np2


def _make_edge_scatter(N, D, CPW, k):
    _, _, Np2 = _acc_rows(N)
    RPT = Np2 // NS
    assert CPW % 6 == 0

    mesh = plsc.VectorSubcoreMesh(core_axis_name="c", subcore_axis_name="s")
    scratch = (
        [pltpu.VMEM((3, k), jnp.int32)] * 3 +
        [pltpu.VMEM((k, D), F32)] * 2 +
        [pltpu.VMEM((k, D), F32)] * 2 +
        [pltpu.VMEM_SHARED((Np2, D), F32)] +
        [pltpu.SemaphoreType.DMA] * 3 +
        [pltpu.SemaphoreType.DMA] * 2
    )

    def body(eidx, xtab, rtab, zpre, out_pre,
             idx0, idx1, idx2, xr0, xr1, rr0, rr1, pre_sh,
             si0, si1, si2, sg0, sg1):
        idx = (idx0, idx1, idx2)
        xr = (xr0, xr1)
        rr = (rr0, rr1)
        si = (si0, si1, si2)
        sg = (sg0, sg1)
        c = lax.axis_index("c")
        s = lax.axis_index("s")
        base = (c * NS + s) * CPW
        rows = pl.ds(s * RPT, RPT)

        pltpu.sync_copy(zpre, pre_sh.at[rows])

        def issue_idx(j, t):
            pltpu.async_copy(eidx.at[base + j], idx[t], si[t])

        def wait_idx(t):
            pltpu.make_async_copy(eidx.at[base], idx[t], si[t]).wait()

        def issue_gather(t, b):
            pltpu.async_copy(xtab.at[idx[t].at[0]], xr[b], sg[b])
            pltpu.async_copy(rtab.at[idx[t].at[2]], rr[b], sg[b])

        def wait_gather(t, b):
            pltpu.make_async_copy(xtab.at[idx[t].at[0]], xr[b], sg[b]).wait()
            pltpu.make_async_copy(rtab.at[idx[t].at[2]], rr[b], sg[b]).wait()

        issue_idx(0, 0)
        issue_idx(1, 1)
        wait_idx(0)
        issue_gather(0, 0)
        plsc.subcore_barrier()

        def superstep(jj, carry):
            for u in range(6):
                j = jj * 6 + u
                b = u % 2
                t = u % 3
                tn = (u + 1) % 3
                t2 = (u + 2) % 3

                @pl.when(j + 1 < CPW)
                def _():
                    wait_idx(tn)
                    issue_gather(tn, 1 - b)

                wait_gather(t, b)

                @pl.when(j + 2 < CPW)
                def _():
                    issue_idx(j + 2, t2)

                def edge(i, carry2):
                    for q in range(D // L):
                        sl = pl.ds(q * L, L)
                        xr[b][i, sl] = xr[b][i, sl] * rr[b][i, sl]
                    return carry2

                lax.fori_loop(0, k, edge, 0)
                pltpu.sync_copy(xr[b], pre_sh.at[idx[t].at[1]], add=True)
            return carry

        lax.fori_loop(0, CPW // 6, superstep, 0)
        plsc.subcore_barrier()
        pltpu.sync_copy(pre_sh.at[rows], out_pre.at[c].at[rows])

    return pl.kernel(body, out_type=jax.ShapeDtypeStruct((NC, Np2, D), F32),
                     mesh=mesh, scratch_types=tuple(scratch))


def _make_deg_scatter(N, D, CPT, k2):
    _, _, Np2 = _acc_rows(N)
    RPT = Np2 // NS

    mesh = plsc.VectorSubcoreMesh(core_axis_name="c", subcore_axis_name="s")
    scratch = [
        pltpu.VMEM((k2,), jnp.int32),
        pltpu.VMEM((k2, D), F32),
        pltpu.VMEM_SHARED((Np2, D), F32),
    ]

    def body(dst_hbm, zpre, out, dstv, ones_v, acc, ):
        c = lax.axis_index("c")
        s = lax.axis_index("s")
        wid = c * NS + s
        rows = pl.ds(s * RPT, RPT)
        ones = jnp.full((L,), 1.0, F32)

        pltpu.sync_copy(zpre, acc.at[rows])

        def orow(i, carry):
            for q in range(D // L):
                ones_v[i, pl.ds(q * L, L)] = ones
            return carry

        lax.fori_loop(0, k2, orow, 0)
        plsc.subcore_barrier()

        def chunk(j, carry):
            pltpu.sync_copy(dst_hbm.at[pl.ds((wid * CPT + j) * k2, k2)], dstv)
            pltpu.sync_copy(ones_v, acc.at[dstv], add=True)
            return carry

        lax.fori_loop(0, CPT, chunk, 0)
        plsc.subcore_barrier()
        pltpu.sync_copy(acc.at[rows], out.at[c].at[rows])

    return pl.kernel(body, out_type=jax.ShapeDtypeStruct((NC, Np2, D), F32),
                     mesh=mesh, scratch_types=tuple(scratch))


def _time_body(tin, Wt, bt, g, b, out):
    y = jnp.dot(tin[...], Wt[...], preferred_element_type=F32) + bt[...]
    out[...] = _ln(y, g[...], b[...])


def _rel_body(r, Wr1, gr1, br1, Wr2, gr2, br2, r1_out, r2_out):
    r1 = jnp.tanh(_ln(jnp.dot(r[...], Wr1[...], preferred_element_type=F32),
                      gr1[...], br1[...]))
    r1_out[...] = r1
    r2_out[...] = jnp.tanh(_ln(jnp.dot(r1, Wr2[...], preferred_element_type=F32),
                               gr2[...], br2[...]))


def _layer_body(pre0, pre1, dg0, dg1, x, t, W, Wl, b, g, be, out):
    pre = pre0[...] + pre1[...]
    deg = (dg0[...] + dg1[...])[:, :1]
    agg = jnp.dot(pre, W[...], preferred_element_type=F32)
    agg = agg / jnp.maximum(deg, 1.0)
    y = agg + jnp.dot(x[...], Wl[...], preferred_element_type=F32)
    y = y + b[...] + t[...]
    out[...] = jnp.tanh(_ln(y, g[...], be[...]))


def _row_spec(bt, d):
    return pl.BlockSpec((bt, d), lambda i: (i, 0))


def _full_spec(shape):
    return pl.BlockSpec(shape, lambda i: tuple(0 for _ in shape))


def _pad_edges(srcv, dstv, etv, N, k):
    E2 = srcv.shape[0]
    cpw = 6 * (-(-E2 // (NW * k * 6)))
    ep = NW * cpw * k
    src = jnp.pad(srcv, (0, ep - E2)).reshape(NW * cpw, 1, k)
    dstp = jnp.pad(dstv, (0, ep - E2), constant_values=N).reshape(NW * cpw, 1, k)
    et = jnp.pad(etv, (0, ep - E2)).reshape(NW * cpw, 1, k)
    return jnp.concatenate([src, dstp, et], axis=1), cpw


def kernel(edge_index, edge_type, time_emds, ent_emds, rel_emds, Wt, bt, g_t,
           b_tl, W1, Wl1, Wr1, b1, ge1, be1, gr1, br1, W2, Wl2, Wr2, b2, ge2,
           be2, gr2, br2):
    N, D = ent_emds.shape
    E = edge_type.shape[0]
    R = rel_emds.shape[0]
    DEGBASE, DEGROWS, Np2 = _acc_rows(N)

    src = edge_index[0]
    dst = edge_index[1]

    eidx1, CPW1 = _pad_edges(src, dst, edge_type, N, K)
    K2 = 256
    CPT = -(-E // (NW * K2))
    dstpad = jnp.pad(dst, (0, NW * CPT * K2 - E), constant_values=N)
    zpre = jnp.zeros((Np2 // NS, D), F32)

    BT = 2000
    grid = N // BT
    t = pl.pallas_call(
        _time_body,
        grid=(grid,),
        in_specs=[_row_spec(BT, D), _full_spec((D, D)), _full_spec((1, D)),
                  _full_spec((1, D)), _full_spec((1, D))],
        out_specs=_row_spec(BT, D),
        out_shape=jax.ShapeDtypeStruct((N, D), F32),
    )(time_emds, Wt, bt.reshape(1, D), g_t.reshape(1, D), b_tl.reshape(1, D))

    r1, r2 = pl.pallas_call(
        _rel_body,
        in_specs=[pl.BlockSpec((R, D), lambda: (0, 0)),
                  pl.BlockSpec((D, D), lambda: (0, 0))] +
                 [pl.BlockSpec((1, D), lambda: (0, 0))] * 2 +
                 [pl.BlockSpec((D, D), lambda: (0, 0))] +
                 [pl.BlockSpec((1, D), lambda: (0, 0))] * 2,
        out_specs=[pl.BlockSpec((R, D), lambda: (0, 0))] * 2,
        out_shape=[jax.ShapeDtypeStruct((R, D), F32)] * 2,
    )(rel_emds, Wr1, gr1.reshape(1, D), br1.reshape(1, D),
      Wr2, gr2.reshape(1, D), br2.reshape(1, D))

    layer_call = pl.pallas_call(
        _layer_body,
        grid=(grid,),
        in_specs=[_row_spec(BT, D), _row_spec(BT, D),
                  _row_spec(BT, D), _row_spec(BT, D),
                  _row_spec(BT, D), _row_spec(BT, D),
                  _full_spec((D, D)), _full_spec((D, D)),
                  _full_spec((1, D)), _full_spec((1, D)), _full_spec((1, D))],
        out_specs=_row_spec(BT, D),
        out_shape=jax.ShapeDtypeStruct((N, D), F32),
    )

    deg_call = _make_deg_scatter(N, D, CPT, K2)
    dacc = deg_call(dstpad, zpre)

    scat = _make_edge_scatter(N, D, CPW1, K)
    eidx1b, dacc = lax.optimization_barrier((eidx1, dacc))
    acc1 = scat(eidx1b, ent_emds, rel_emds, zpre)
    x1 = layer_call(acc1[0], acc1[1], dacc[0], dacc[1], ent_emds, t, W1, Wl1,
                    b1.reshape(1, D), ge1.reshape(1, D), be1.reshape(1, D))

    acc2 = scat(eidx1, x1, r1, zpre)
    x2 = layer_call(acc2[0], acc2[1], dacc[0], dacc[1], x1, t, W2, Wl2,
                    b2.reshape(1, D), ge2.reshape(1, D), be2.reshape(1, D))

    return (x2, r2)
